# baseline (speedup 1.0000x reference)
import functools

import jax
import jax.numpy as jnp
from jax.experimental import pallas as pl
from jax.experimental.pallas import tpu as pltpu

D_MODEL = 1024
D_FF = 2048
NUM_E = 8
ROW_TILE = 256

_INTERPRET = False


def _silu(v):
    return v * (1.0 / (1.0 + jnp.exp(-v)))


def _dense_moe_body(x_ref, w1_ref, w2_ref, w3_ref, gate_ref, out_ref, comb_ref):
    e = pl.program_id(1)
    x = x_ref[...]

    @pl.when(e == 0)
    def _():
        logits = jnp.dot(x, gate_ref[...], preferred_element_type=jnp.float32)
        iota = jax.lax.broadcasted_iota(jnp.int32, (ROW_TILE, NUM_E), 1)
        m0 = jnp.max(logits, axis=1, keepdims=True)
        a0 = jnp.min(jnp.where(logits == m0, iota, NUM_E), axis=1, keepdims=True)
        masked = jnp.where(iota == a0, jnp.float32(-1e30), logits)
        m1 = jnp.max(masked, axis=1, keepdims=True)
        a1 = jnp.min(jnp.where(masked == m1, iota, NUM_E), axis=1, keepdims=True)
        t = jnp.exp(m1 - m0)
        w0 = 1.0 / (1.0 + t)
        w1 = t / (1.0 + t)
        comb_ref[...] = (jnp.where(iota == a0, w0, 0.0)
                         + jnp.where(iota == a1, w1, 0.0))

    w1 = w1_ref[0]
    w2 = w2_ref[0]
    w3 = w3_ref[0]
    gate = _silu(jnp.dot(x, w1, preferred_element_type=jnp.float32))
    up = jnp.dot(x, w2, preferred_element_type=jnp.float32)
    eout = jnp.dot(gate * up, w3, preferred_element_type=jnp.float32)

    iota = jax.lax.broadcasted_iota(jnp.int32, (ROW_TILE, NUM_E), 1)
    w_col = jnp.sum(jnp.where(iota == e, comb_ref[...], 0.0), axis=1, keepdims=True)
    w_col = jnp.where(e == NUM_E, jnp.float32(1.0), w_col)
    contrib = eout * w_col

    @pl.when(e == 0)
    def _():
        out_ref[...] = contrib

    @pl.when(e != 0)
    def _():
        out_ref[...] = out_ref[...] + contrib


def kernel(x, shared_w1, shared_w2, shared_w3, expert_w1, expert_w2, expert_w3, gate_w):
    Bn, Tn, C = x.shape
    n = Bn * Tn
    x_flat = x.reshape(n, C)
    ws1 = jnp.concatenate([expert_w1, shared_w1[None]], axis=0)
    ws2 = jnp.concatenate([expert_w2, shared_w2[None]], axis=0)
    ws3 = jnp.concatenate([expert_w3, shared_w3[None]], axis=0)
    n_tiles = n // ROW_TILE

    out = pl.pallas_call(
        _dense_moe_body,
        grid=(n_tiles, NUM_E + 1),
        in_specs=[
            pl.BlockSpec((ROW_TILE, D_MODEL), lambda t, e: (t, 0)),
            pl.BlockSpec((1, D_MODEL, D_FF), lambda t, e: (e, 0, 0)),
            pl.BlockSpec((1, D_MODEL, D_FF), lambda t, e: (e, 0, 0)),
            pl.BlockSpec((1, D_FF, D_MODEL), lambda t, e: (e, 0, 0)),
            pl.BlockSpec((D_MODEL, NUM_E), lambda t, e: (0, 0)),
        ],
        out_specs=pl.BlockSpec((ROW_TILE, D_MODEL), lambda t, e: (t, 0)),
        out_shape=jax.ShapeDtypeStruct((n, D_MODEL), jnp.float32),
        scratch_shapes=[pltpu.VMEM((ROW_TILE, NUM_E), jnp.float32)],
        compiler_params=pltpu.CompilerParams(
            dimension_semantics=("arbitrary", "arbitrary"),
        ),
        interpret=_INTERPRET,
    )(x_flat, ws1, ws2, ws3, gate_w)

    final_out = out.reshape(Bn, Tn, C)
    aux_loss = jnp.array(0.0, dtype=jnp.float32)
    return (final_out, aux_loss)


# speedup vs baseline: 3.2075x; 3.2075x over previous
import functools

import jax
import jax.numpy as jnp
from jax import lax
from jax.experimental import pallas as pl
from jax.experimental.pallas import tpu as pltpu
from jax.experimental.pallas import tpu_sc as plsc

D_MODEL = 1024
D_FF = 2048
NUM_E = 8
ROW_TILE = 256

NC = 2
NS = 16
NW = NC * NS

_INTERPRET = False


def _silu(v):
    return v * (1.0 / (1.0 + jnp.exp(-v)))


def _router_shared_body(x_ref, w1_ref, w2_ref, w3_ref, gate_ref,
                        sh_ref, a0_ref, a1_ref, w0_ref, w1c_ref):
    x = x_ref[...]
    rows = x.shape[0]
    logits = jnp.dot(x, gate_ref[...], preferred_element_type=jnp.float32)
    iota = lax.broadcasted_iota(jnp.int32, (rows, NUM_E), 1)
    m0 = jnp.max(logits, axis=1, keepdims=True)
    a0 = jnp.min(jnp.where(logits == m0, iota, NUM_E), axis=1, keepdims=True)
    masked = jnp.where(iota == a0, jnp.float32(-1e30), logits)
    m1 = jnp.max(masked, axis=1, keepdims=True)
    a1 = jnp.min(jnp.where(masked == m1, iota, NUM_E), axis=1, keepdims=True)
    t = jnp.exp(m1 - m0)
    a0_ref[...] = a0
    a1_ref[...] = a1
    w0_ref[...] = 1.0 / (1.0 + t)
    w1c_ref[...] = t / (1.0 + t)

    gate = _silu(jnp.dot(x, w1_ref[...], preferred_element_type=jnp.float32))
    up = jnp.dot(x, w2_ref[...], preferred_element_type=jnp.float32)
    sh_ref[...] = jnp.dot(gate * up, w3_ref[...], preferred_element_type=jnp.float32)


def _router_shared(x_flat, sw1, sw2, sw3, gate_w):
    n = x_flat.shape[0]
    n_tiles = n // ROW_TILE
    return pl.pallas_call(
        _router_shared_body,
        grid=(n_tiles,),
        in_specs=[
            pl.BlockSpec((ROW_TILE, D_MODEL), lambda t: (t, 0)),
            pl.BlockSpec((D_MODEL, D_FF), lambda t: (0, 0)),
            pl.BlockSpec((D_MODEL, D_FF), lambda t: (0, 0)),
            pl.BlockSpec((D_FF, D_MODEL), lambda t: (0, 0)),
            pl.BlockSpec((D_MODEL, NUM_E), lambda t: (0, 0)),
        ],
        out_specs=[
            pl.BlockSpec((ROW_TILE, D_MODEL), lambda t: (t, 0)),
            pl.BlockSpec((ROW_TILE, 1), lambda t: (t, 0)),
            pl.BlockSpec((ROW_TILE, 1), lambda t: (t, 0)),
            pl.BlockSpec((ROW_TILE, 1), lambda t: (t, 0)),
            pl.BlockSpec((ROW_TILE, 1), lambda t: (t, 0)),
        ],
        out_shape=[
            jax.ShapeDtypeStruct((n, D_MODEL), jnp.float32),
            jax.ShapeDtypeStruct((n, 1), jnp.int32),
            jax.ShapeDtypeStruct((n, 1), jnp.int32),
            jax.ShapeDtypeStruct((n, 1), jnp.float32),
            jax.ShapeDtypeStruct((n, 1), jnp.float32),
        ],
        compiler_params=pltpu.CompilerParams(
            dimension_semantics=("arbitrary",),
        ),
        interpret=_INTERPRET,
    )(x_flat, sw1, sw2, sw3, gate_w)


def _wid():
    return lax.axis_index("s") * NC + lax.axis_index("c")


def _splat(v, k):
    return v[jnp.full((16,), k, jnp.int32)]


def _cumsum16(v, iota):
    for k in (1, 2, 4, 8):
        sv = v[jnp.maximum(iota - k, 0)]
        v = v + jnp.where(iota >= k, sv, 0)
    return v


def _make_hist_kernel(n_assign):
    chunk = n_assign // NW
    mesh = plsc.VectorSubcoreMesh(core_axis_name="c", subcore_axis_name="s")

    @functools.partial(
        pl.kernel,
        out_type=jax.ShapeDtypeStruct((NW, 16), jnp.int32),
        mesh=mesh,
        scratch_types=[
            pltpu.VMEM((chunk,), jnp.int32),
            pltpu.VMEM((16,), jnp.int32),
        ],
    )
    def hist_kernel(e_hbm, hist_hbm, ids_v, hist_v):
        w = _wid()
        pltpu.sync_copy(e_hbm.at[pl.ds(w * chunk, chunk)], ids_v)
        iota = lax.iota(jnp.int32, 16)
        accs = [jnp.zeros((16,), jnp.int32) for _ in range(NUM_E)]
        for j in range(chunk // 16):
            v = ids_v[pl.ds(j * 16, 16)]
            for e in range(NUM_E):
                accs[e] = accs[e] + jnp.where(v == e, 1, 0)
        hist = jnp.zeros((16,), jnp.int32)
        for e in range(NUM_E):
            tot_e = _splat(_cumsum16(accs[e], iota), 15)
            hist = jnp.where(iota == e, tot_e, hist)
        hist_v[...] = hist
        pltpu.sync_copy(hist_v, hist_hbm.at[w])

    return hist_kernel


def _make_sort_kernel(n_assign, te_len):
    chunk = n_assign // NW
    mesh = plsc.VectorSubcoreMesh(core_axis_name="c", subcore_axis_name="s")

    @functools.partial(
        pl.kernel,
        out_type=[
            jax.ShapeDtypeStruct((n_assign,), jnp.int32),
            jax.ShapeDtypeStruct((te_len,), jnp.int32),
        ],
        mesh=mesh,
        scratch_types=[
            pltpu.VMEM((chunk,), jnp.int32),
            pltpu.VMEM((NW, 16), jnp.int32),
            pltpu.VMEM((chunk,), jnp.int32),
            pltpu.VMEM((te_len,), jnp.int32),
        ],
    )
    def sort_kernel(e_hbm, hist_hbm, dst_hbm, te_hbm, ids_v, allh_v, dst_v, te_v):
        w = _wid()
        pltpu.sync_copy(e_hbm.at[pl.ds(w * chunk, chunk)], ids_v)
        pltpu.sync_copy(hist_hbm, allh_v)
        iota = lax.iota(jnp.int32, 16)
        tot = jnp.zeros((16,), jnp.int32)
        pre = jnp.zeros((16,), jnp.int32)
        for ww in range(NW):
            row = allh_v[ww]
            tot = tot + row
            pre = pre + row * jnp.where(ww < w, 1, 0)
        padded = (tot + (ROW_TILE - 1)) & jnp.int32(-ROW_TILE)
        csum = _cumsum16(padded, iota)
        off = csum - padded
        start = off + pre

        cnt = jnp.zeros((16,), jnp.int32)
        for j in range(chunk // 16):
            v = ids_v[pl.ds(j * 16, 16)]
            rank = jnp.zeros((16,), jnp.int32)
            for k in range(1, 16):
                sv = v[jnp.maximum(iota - k, 0)]
                rank = rank + jnp.where((iota >= k) & (sv == v), 1, 0)
            dst_v[pl.ds(j * 16, 16)] = start[v] + cnt[v] + rank
            cc = jnp.zeros((16,), jnp.int32)
            for k in range(16):
                cc = cc + jnp.where(iota == _splat(v, k), 1, 0)
            cnt = cnt + cc
        pltpu.sync_copy(dst_v, dst_hbm.at[pl.ds(w * chunk, chunk)])

        @pl.when(w == 0)
        def _():
            for g in range(te_len // 16):
                tstart = (iota + g * 16) * ROW_TILE
                acc = jnp.zeros((16,), jnp.int32)
                for e in range(NUM_E):
                    acc = acc + jnp.where(_splat(off, e) <= tstart, 1, 0)
                te_v[pl.ds(g * 16, 16)] = acc - 1
            pltpu.sync_copy(te_v, te_hbm)

    return sort_kernel


def _make_dispatch_kernel(n_tok, pad_total):
    per_w = n_tok // NW
    rows = 32
    mesh = plsc.VectorSubcoreMesh(core_axis_name="c", subcore_axis_name="s")

    @functools.partial(
        pl.kernel,
        out_type=jax.ShapeDtypeStruct((pad_total, D_MODEL), jnp.float32),
        mesh=mesh,
        scratch_types=[
            pltpu.VMEM((rows, D_MODEL), jnp.float32),
            pltpu.VMEM((rows,), jnp.int32),
            pltpu.VMEM((rows,), jnp.int32),
            pltpu.SemaphoreType.DMA,
        ],
    )
    def dispatch_kernel(x_hbm, dst_hbm, disp_hbm, rows_v, idx0_v, idx1_v, sem):
        w = _wid()
        for c in range(per_w // rows):
            tb = w * per_w + c * rows
            pltpu.sync_copy(x_hbm.at[pl.ds(tb, rows)], rows_v)
            pltpu.sync_copy(dst_hbm.at[pl.ds(tb, rows)], idx0_v)
            pltpu.sync_copy(dst_hbm.at[pl.ds(n_tok + tb, rows)], idx1_v)
            cp0 = pltpu.async_copy(rows_v, disp_hbm.at[idx0_v], sem)
            cp1 = pltpu.async_copy(rows_v, disp_hbm.at[idx1_v], sem)
            cp0.wait()
            cp1.wait()

    return dispatch_kernel


def _expert_ffn_body(te_ref, x_ref, w1_ref, w2_ref, w3_ref, out_ref):
    del te_ref
    x = x_ref[...]
    gate = _silu(jnp.dot(x, w1_ref[0], preferred_element_type=jnp.float32))
    up = jnp.dot(x, w2_ref[0], preferred_element_type=jnp.float32)
    out_ref[...] = jnp.dot(gate * up, w3_ref[0], preferred_element_type=jnp.float32)


def _expert_ffn(te, disp, ew1, ew2, ew3, n_tiles):
    pad_total = disp.shape[0]
    grid_spec = pltpu.PrefetchScalarGridSpec(
        num_scalar_prefetch=1,
        grid=(n_tiles,),
        in_specs=[
            pl.BlockSpec((ROW_TILE, D_MODEL), lambda t, te: (t, 0)),
            pl.BlockSpec((1, D_MODEL, D_FF), lambda t, te: (te[t], 0, 0)),
            pl.BlockSpec((1, D_MODEL, D_FF), lambda t, te: (te[t], 0, 0)),
            pl.BlockSpec((1, D_FF, D_MODEL), lambda t, te: (te[t], 0, 0)),
        ],
        out_specs=pl.BlockSpec((ROW_TILE, D_MODEL), lambda t, te: (t, 0)),
    )
    return pl.pallas_call(
        _expert_ffn_body,
        grid_spec=grid_spec,
        out_shape=jax.ShapeDtypeStruct((pad_total, D_MODEL), jnp.float32),
        compiler_params=pltpu.CompilerParams(
            dimension_semantics=("arbitrary",),
        ),
        interpret=_INTERPRET,
    )(te, disp, ew1, ew2, ew3)


def _make_gather_kernel(n_assign, pad_total):
    per_w = n_assign // NW
    rows = 32
    mesh = plsc.VectorSubcoreMesh(core_axis_name="c", subcore_axis_name="s")

    @functools.partial(
        pl.kernel,
        out_type=jax.ShapeDtypeStruct((n_assign, D_MODEL), jnp.float32),
        mesh=mesh,
        scratch_types=[
            pltpu.VMEM((rows, D_MODEL), jnp.float32),
            pltpu.VMEM((rows,), jnp.int32),
            pltpu.SemaphoreType.DMA,
        ],
    )
    def gather_kernel(eout_hbm, dst_hbm, g_hbm, rows_v, idx_v, sem):
        w = _wid()
        for c in range(per_w // rows):
            ab = w * per_w + c * rows
            pltpu.sync_copy(dst_hbm.at[pl.ds(ab, rows)], idx_v)
            pltpu.async_copy(eout_hbm.at[idx_v], rows_v, sem).wait()
            pltpu.sync_copy(rows_v, g_hbm.at[pl.ds(ab, rows)])

    return gather_kernel


def _combine_body(sh_ref, g0_ref, g1_ref, w0_ref, w1_ref, out_ref):
    out_ref[...] = (sh_ref[...]
                    + w0_ref[...] * g0_ref[...]
                    + w1_ref[...] * g1_ref[...])


def _combine(shared_out, g, w0c, w1c, n_tok):
    n_tiles = n_tok // ROW_TILE
    return pl.pallas_call(
        _combine_body,
        grid=(n_tiles,),
        in_specs=[
            pl.BlockSpec((ROW_TILE, D_MODEL), lambda t: (t, 0)),
            pl.BlockSpec((ROW_TILE, D_MODEL), lambda t: (t, 0)),
            pl.BlockSpec((ROW_TILE, D_MODEL), lambda t: (t + n_tok // ROW_TILE, 0)),
            pl.BlockSpec((ROW_TILE, 1), lambda t: (t, 0)),
            pl.BlockSpec((ROW_TILE, 1), lambda t: (t, 0)),
        ],
        out_specs=pl.BlockSpec((ROW_TILE, D_MODEL), lambda t: (t, 0)),
        out_shape=jax.ShapeDtypeStruct((n_tok, D_MODEL), jnp.float32),
        compiler_params=pltpu.CompilerParams(
            dimension_semantics=("arbitrary",),
        ),
        interpret=_INTERPRET,
    )(shared_out, g, g, w0c, w1c)


def kernel(x, shared_w1, shared_w2, shared_w3, expert_w1, expert_w2, expert_w3, gate_w):
    Bn, Tn, C = x.shape
    n_tok = Bn * Tn
    n_assign = 2 * n_tok
    pad_total = n_assign + NUM_E * ROW_TILE
    n_tiles = n_assign // ROW_TILE + NUM_E
    te_len = 64

    x_flat = x.reshape(n_tok, C)

    shared_out, a0, a1, w0c, w1c = _router_shared(
        x_flat, shared_w1, shared_w2, shared_w3, gate_w)
    e_all = jnp.concatenate([a0.reshape(-1), a1.reshape(-1)])

    hist = _make_hist_kernel(n_assign)(e_all)
    dst_all, te = _make_sort_kernel(n_assign, te_len)(e_all, hist)
    disp = _make_dispatch_kernel(n_tok, pad_total)(x_flat, dst_all)
    eout = _expert_ffn(te, disp, expert_w1, expert_w2, expert_w3, n_tiles)
    g = _make_gather_kernel(n_assign, pad_total)(eout, dst_all)
    out = _combine(shared_out, g, w0c, w1c, n_tok)

    final_out = out.reshape(Bn, Tn, C)
    aux_loss = jnp.array(0.0, dtype=jnp.float32)
    return (final_out, aux_loss)


# speedup vs baseline: 3.2135x; 1.0019x over previous
import functools

import jax
import jax.numpy as jnp
from jax import lax
from jax.experimental import pallas as pl
from jax.experimental.pallas import tpu as pltpu
from jax.experimental.pallas import tpu_sc as plsc

D_MODEL = 1024
D_FF = 2048
NUM_E = 8
ROW_TILE = 256

NC = 2
NS = 16
NW = NC * NS

_INTERPRET = False


def _silu(v):
    return v * (1.0 / (1.0 + jnp.exp(-v)))


def _router_shared_body(x_ref, w1_ref, w2_ref, w3_ref, gate_ref,
                        sh_ref, a0_ref, a1_ref, w0_ref, w1c_ref):
    x = x_ref[...]
    rows = x.shape[0]
    logits = jnp.dot(x, gate_ref[...], preferred_element_type=jnp.float32)
    iota = lax.broadcasted_iota(jnp.int32, (rows, NUM_E), 1)
    m0 = jnp.max(logits, axis=1, keepdims=True)
    a0 = jnp.min(jnp.where(logits == m0, iota, NUM_E), axis=1, keepdims=True)
    masked = jnp.where(iota == a0, jnp.float32(-1e30), logits)
    m1 = jnp.max(masked, axis=1, keepdims=True)
    a1 = jnp.min(jnp.where(masked == m1, iota, NUM_E), axis=1, keepdims=True)
    t = jnp.exp(m1 - m0)
    a0_ref[...] = a0
    a1_ref[...] = a1
    w0_ref[...] = 1.0 / (1.0 + t)
    w1c_ref[...] = t / (1.0 + t)

    xb = x.astype(jnp.bfloat16)
    gate = _silu(jnp.dot(xb, w1_ref[...].astype(jnp.bfloat16),
                         preferred_element_type=jnp.float32))
    up = jnp.dot(xb, w2_ref[...].astype(jnp.bfloat16),
                 preferred_element_type=jnp.float32)
    sh_ref[...] = jnp.dot((gate * up).astype(jnp.bfloat16),
                          w3_ref[...].astype(jnp.bfloat16),
                          preferred_element_type=jnp.float32)


def _router_shared(x_flat, sw1, sw2, sw3, gate_w):
    n = x_flat.shape[0]
    n_tiles = n // ROW_TILE
    return pl.pallas_call(
        _router_shared_body,
        grid=(n_tiles,),
        in_specs=[
            pl.BlockSpec((ROW_TILE, D_MODEL), lambda t: (t, 0)),
            pl.BlockSpec((D_MODEL, D_FF), lambda t: (0, 0)),
            pl.BlockSpec((D_MODEL, D_FF), lambda t: (0, 0)),
            pl.BlockSpec((D_FF, D_MODEL), lambda t: (0, 0)),
            pl.BlockSpec((D_MODEL, NUM_E), lambda t: (0, 0)),
        ],
        out_specs=[
            pl.BlockSpec((ROW_TILE, D_MODEL), lambda t: (t, 0)),
            pl.BlockSpec((ROW_TILE, 1), lambda t: (t, 0)),
            pl.BlockSpec((ROW_TILE, 1), lambda t: (t, 0)),
            pl.BlockSpec((ROW_TILE, 1), lambda t: (t, 0)),
            pl.BlockSpec((ROW_TILE, 1), lambda t: (t, 0)),
        ],
        out_shape=[
            jax.ShapeDtypeStruct((n, D_MODEL), jnp.float32),
            jax.ShapeDtypeStruct((n, 1), jnp.int32),
            jax.ShapeDtypeStruct((n, 1), jnp.int32),
            jax.ShapeDtypeStruct((n, 1), jnp.float32),
            jax.ShapeDtypeStruct((n, 1), jnp.float32),
        ],
        compiler_params=pltpu.CompilerParams(
            dimension_semantics=("arbitrary",),
        ),
        interpret=_INTERPRET,
    )(x_flat, sw1, sw2, sw3, gate_w)


def _wid():
    return lax.axis_index("s") * NC + lax.axis_index("c")


def _splat(v, k):
    return v[jnp.full((16,), k, jnp.int32)]


def _cumsum16(v, iota):
    for k in (1, 2, 4, 8):
        sv = v[jnp.maximum(iota - k, 0)]
        v = v + jnp.where(iota >= k, sv, 0)
    return v


def _make_hist_kernel(n_assign):
    chunk = n_assign // NW
    mesh = plsc.VectorSubcoreMesh(core_axis_name="c", subcore_axis_name="s")

    @functools.partial(
        pl.kernel,
        out_type=jax.ShapeDtypeStruct((NW, 16), jnp.int32),
        mesh=mesh,
        scratch_types=[
            pltpu.VMEM((chunk,), jnp.int32),
            pltpu.VMEM((16,), jnp.int32),
        ],
    )
    def hist_kernel(e_hbm, hist_hbm, ids_v, hist_v):
        w = _wid()
        pltpu.sync_copy(e_hbm.at[pl.ds(w * chunk, chunk)], ids_v)
        iota = lax.iota(jnp.int32, 16)
        accs = [jnp.zeros((16,), jnp.int32) for _ in range(NUM_E)]
        for j in range(chunk // 16):
            v = ids_v[pl.ds(j * 16, 16)]
            for e in range(NUM_E):
                accs[e] = accs[e] + jnp.where(v == e, 1, 0)
        hist = jnp.zeros((16,), jnp.int32)
        for e in range(NUM_E):
            tot_e = _splat(_cumsum16(accs[e], iota), 15)
            hist = jnp.where(iota == e, tot_e, hist)
        hist_v[...] = hist
        pltpu.sync_copy(hist_v, hist_hbm.at[w])

    return hist_kernel


def _make_sort_kernel(n_assign, te_len):
    chunk = n_assign // NW
    mesh = plsc.VectorSubcoreMesh(core_axis_name="c", subcore_axis_name="s")

    @functools.partial(
        pl.kernel,
        out_type=[
            jax.ShapeDtypeStruct((n_assign,), jnp.int32),
            jax.ShapeDtypeStruct((te_len,), jnp.int32),
        ],
        mesh=mesh,
        scratch_types=[
            pltpu.VMEM((chunk,), jnp.int32),
            pltpu.VMEM((NW, 16), jnp.int32),
            pltpu.VMEM((chunk,), jnp.int32),
            pltpu.VMEM((te_len,), jnp.int32),
        ],
    )
    def sort_kernel(e_hbm, hist_hbm, dst_hbm, te_hbm, ids_v, allh_v, dst_v, te_v):
        w = _wid()
        pltpu.sync_copy(e_hbm.at[pl.ds(w * chunk, chunk)], ids_v)
        pltpu.sync_copy(hist_hbm, allh_v)
        iota = lax.iota(jnp.int32, 16)
        tot = jnp.zeros((16,), jnp.int32)
        pre = jnp.zeros((16,), jnp.int32)
        for ww in range(NW):
            row = allh_v[ww]
            tot = tot + row
            pre = pre + row * jnp.where(ww < w, 1, 0)
        padded = (tot + (ROW_TILE - 1)) & jnp.int32(-ROW_TILE)
        csum = _cumsum16(padded, iota)
        off = csum - padded
        start = off + pre

        cnt = jnp.zeros((16,), jnp.int32)
        for j in range(chunk // 16):
            v = ids_v[pl.ds(j * 16, 16)]
            rank = jnp.zeros((16,), jnp.int32)
            for k in range(1, 16):
                sv = v[jnp.maximum(iota - k, 0)]
                rank = rank + jnp.where((iota >= k) & (sv == v), 1, 0)
            dst_v[pl.ds(j * 16, 16)] = start[v] + cnt[v] + rank
            cc = jnp.zeros((16,), jnp.int32)
            for k in range(16):
                cc = cc + jnp.where(iota == _splat(v, k), 1, 0)
            cnt = cnt + cc
        pltpu.sync_copy(dst_v, dst_hbm.at[pl.ds(w * chunk, chunk)])

        @pl.when(w == 0)
        def _():
            for g in range(te_len // 16):
                tstart = (iota + g * 16) * ROW_TILE
                acc = jnp.zeros((16,), jnp.int32)
                for e in range(NUM_E):
                    acc = acc + jnp.where(_splat(off, e) <= tstart, 1, 0)
                te_v[pl.ds(g * 16, 16)] = acc - 1
            pltpu.sync_copy(te_v, te_hbm)

    return sort_kernel


def _make_dispatch_kernel(n_tok, pad_total):
    per_w = n_tok // NW
    rows = 32
    mesh = plsc.VectorSubcoreMesh(core_axis_name="c", subcore_axis_name="s")

    @functools.partial(
        pl.kernel,
        out_type=jax.ShapeDtypeStruct((pad_total, D_MODEL), jnp.float32),
        mesh=mesh,
        scratch_types=[
            pltpu.VMEM((rows, D_MODEL), jnp.float32),
            pltpu.VMEM((rows,), jnp.int32),
            pltpu.VMEM((rows,), jnp.int32),
            pltpu.SemaphoreType.DMA,
        ],
    )
    def dispatch_kernel(x_hbm, dst_hbm, disp_hbm, rows_v, idx0_v, idx1_v, sem):
        w = _wid()
        for c in range(per_w // rows):
            tb = w * per_w + c * rows
            pltpu.sync_copy(x_hbm.at[pl.ds(tb, rows)], rows_v)
            pltpu.sync_copy(dst_hbm.at[pl.ds(tb, rows)], idx0_v)
            pltpu.sync_copy(dst_hbm.at[pl.ds(n_tok + tb, rows)], idx1_v)
            cp0 = pltpu.async_copy(rows_v, disp_hbm.at[idx0_v], sem)
            cp1 = pltpu.async_copy(rows_v, disp_hbm.at[idx1_v], sem)
            cp0.wait()
            cp1.wait()

    return dispatch_kernel


def _expert_ffn_body(te_ref, x_ref, w1_ref, w2_ref, w3_ref, out_ref):
    del te_ref
    xb = x_ref[...].astype(jnp.bfloat16)
    gate = _silu(jnp.dot(xb, w1_ref[0].astype(jnp.bfloat16),
                         preferred_element_type=jnp.float32))
    up = jnp.dot(xb, w2_ref[0].astype(jnp.bfloat16),
                 preferred_element_type=jnp.float32)
    out_ref[...] = jnp.dot((gate * up).astype(jnp.bfloat16),
                           w3_ref[0].astype(jnp.bfloat16),
                           preferred_element_type=jnp.float32)


def _expert_ffn(te, disp, ew1, ew2, ew3, n_tiles):
    pad_total = disp.shape[0]
    grid_spec = pltpu.PrefetchScalarGridSpec(
        num_scalar_prefetch=1,
        grid=(n_tiles,),
        in_specs=[
            pl.BlockSpec((ROW_TILE, D_MODEL), lambda t, te: (t, 0)),
            pl.BlockSpec((1, D_MODEL, D_FF), lambda t, te: (te[t], 0, 0)),
            pl.BlockSpec((1, D_MODEL, D_FF), lambda t, te: (te[t], 0, 0)),
            pl.BlockSpec((1, D_FF, D_MODEL), lambda t, te: (te[t], 0, 0)),
        ],
        out_specs=pl.BlockSpec((ROW_TILE, D_MODEL), lambda t, te: (t, 0)),
    )
    return pl.pallas_call(
        _expert_ffn_body,
        grid_spec=grid_spec,
        out_shape=jax.ShapeDtypeStruct((pad_total, D_MODEL), jnp.float32),
        compiler_params=pltpu.CompilerParams(
            dimension_semantics=("arbitrary",),
        ),
        interpret=_INTERPRET,
    )(te, disp, ew1, ew2, ew3)


def _make_gather_kernel(n_assign, pad_total):
    per_w = n_assign // NW
    rows = 32
    mesh = plsc.VectorSubcoreMesh(core_axis_name="c", subcore_axis_name="s")

    @functools.partial(
        pl.kernel,
        out_type=jax.ShapeDtypeStruct((n_assign, D_MODEL), jnp.float32),
        mesh=mesh,
        scratch_types=[
            pltpu.VMEM((rows, D_MODEL), jnp.float32),
            pltpu.VMEM((rows,), jnp.int32),
            pltpu.SemaphoreType.DMA,
        ],
    )
    def gather_kernel(eout_hbm, dst_hbm, g_hbm, rows_v, idx_v, sem):
        w = _wid()
        for c in range(per_w // rows):
            ab = w * per_w + c * rows
            pltpu.sync_copy(dst_hbm.at[pl.ds(ab, rows)], idx_v)
            pltpu.async_copy(eout_hbm.at[idx_v], rows_v, sem).wait()
            pltpu.sync_copy(rows_v, g_hbm.at[pl.ds(ab, rows)])

    return gather_kernel


def _combine_body(sh_ref, g0_ref, g1_ref, w0_ref, w1_ref, out_ref):
    out_ref[...] = (sh_ref[...]
                    + w0_ref[...] * g0_ref[...]
                    + w1_ref[...] * g1_ref[...])


def _combine(shared_out, g, w0c, w1c, n_tok):
    n_tiles = n_tok // ROW_TILE
    return pl.pallas_call(
        _combine_body,
        grid=(n_tiles,),
        in_specs=[
            pl.BlockSpec((ROW_TILE, D_MODEL), lambda t: (t, 0)),
            pl.BlockSpec((ROW_TILE, D_MODEL), lambda t: (t, 0)),
            pl.BlockSpec((ROW_TILE, D_MODEL), lambda t: (t + n_tok // ROW_TILE, 0)),
            pl.BlockSpec((ROW_TILE, 1), lambda t: (t, 0)),
            pl.BlockSpec((ROW_TILE, 1), lambda t: (t, 0)),
        ],
        out_specs=pl.BlockSpec((ROW_TILE, D_MODEL), lambda t: (t, 0)),
        out_shape=jax.ShapeDtypeStruct((n_tok, D_MODEL), jnp.float32),
        compiler_params=pltpu.CompilerParams(
            dimension_semantics=("arbitrary",),
        ),
        interpret=_INTERPRET,
    )(shared_out, g, g, w0c, w1c)


def kernel(x, shared_w1, shared_w2, shared_w3, expert_w1, expert_w2, expert_w3, gate_w):
    Bn, Tn, C = x.shape
    n_tok = Bn * Tn
    n_assign = 2 * n_tok
    pad_total = n_assign + NUM_E * ROW_TILE
    n_tiles = n_assign // ROW_TILE + NUM_E
    te_len = 64

    x_flat = x.reshape(n_tok, C)

    shared_out, a0, a1, w0c, w1c = _router_shared(
        x_flat, shared_w1, shared_w2, shared_w3, gate_w)
    e_all = jnp.concatenate([a0.reshape(-1), a1.reshape(-1)])

    hist = _make_hist_kernel(n_assign)(e_all)
    dst_all, te = _make_sort_kernel(n_assign, te_len)(e_all, hist)
    disp = _make_dispatch_kernel(n_tok, pad_total)(x_flat, dst_all)
    eout = _expert_ffn(te, disp, expert_w1, expert_w2, expert_w3, n_tiles)
    g = _make_gather_kernel(n_assign, pad_total)(eout, dst_all)
    out = _combine(shared_out, g, w0c, w1c, n_tok)

    final_out = out.reshape(Bn, Tn, C)
    aux_loss = jnp.array(0.0, dtype=jnp.float32)
    return (final_out, aux_loss)


# speedup vs baseline: 3.4027x; 1.0589x over previous
import functools

import jax
import jax.numpy as jnp
from jax import lax
from jax.experimental import pallas as pl
from jax.experimental.pallas import tpu as pltpu
from jax.experimental.pallas import tpu_sc as plsc

D_MODEL = 1024
D_FF = 2048
NUM_E = 8
ROW_TILE = 256

NC = 2
NS = 16
NW = NC * NS

_INTERPRET = False


def _silu(v):
    return v * (1.0 / (1.0 + jnp.exp(-v)))


def _router_body(x_ref, gate_ref, a0_ref, a1_ref, w0_ref, w1c_ref):
    x = x_ref[...]
    rows = x.shape[0]
    logits = jnp.dot(x, gate_ref[...], preferred_element_type=jnp.float32)
    iota = lax.broadcasted_iota(jnp.int32, (rows, NUM_E), 1)
    m0 = jnp.max(logits, axis=1, keepdims=True)
    a0 = jnp.min(jnp.where(logits == m0, iota, NUM_E), axis=1, keepdims=True)
    masked = jnp.where(iota == a0, jnp.float32(-1e30), logits)
    m1 = jnp.max(masked, axis=1, keepdims=True)
    a1 = jnp.min(jnp.where(masked == m1, iota, NUM_E), axis=1, keepdims=True)
    t = jnp.exp(m1 - m0)
    a0_ref[...] = a0
    a1_ref[...] = a1
    w0_ref[...] = 1.0 / (1.0 + t)
    w1c_ref[...] = t / (1.0 + t)


def _router(x_flat, gate_w):
    n = x_flat.shape[0]
    n_tiles = n // ROW_TILE
    return pl.pallas_call(
        _router_body,
        grid=(n_tiles,),
        in_specs=[
            pl.BlockSpec((ROW_TILE, D_MODEL), lambda t: (t, 0)),
            pl.BlockSpec((D_MODEL, NUM_E), lambda t: (0, 0)),
        ],
        out_specs=[
            pl.BlockSpec((ROW_TILE, 1), lambda t: (t, 0)),
            pl.BlockSpec((ROW_TILE, 1), lambda t: (t, 0)),
            pl.BlockSpec((ROW_TILE, 1), lambda t: (t, 0)),
            pl.BlockSpec((ROW_TILE, 1), lambda t: (t, 0)),
        ],
        out_shape=[
            jax.ShapeDtypeStruct((n, 1), jnp.int32),
            jax.ShapeDtypeStruct((n, 1), jnp.int32),
            jax.ShapeDtypeStruct((n, 1), jnp.float32),
            jax.ShapeDtypeStruct((n, 1), jnp.float32),
        ],
        compiler_params=pltpu.CompilerParams(
            dimension_semantics=("arbitrary",),
        ),
        interpret=_INTERPRET,
    )(x_flat, gate_w)


def _shared_body(x_ref, w1_ref, w2_ref, w3_ref, sh_ref):
    x = x_ref[...]
    gate = _silu(jnp.dot(x, w1_ref[...], preferred_element_type=jnp.float32))
    up = jnp.dot(x, w2_ref[...], preferred_element_type=jnp.float32)
    sh_ref[...] = jnp.dot(gate * up, w3_ref[...], preferred_element_type=jnp.float32)


def _shared(x_flat, sw1, sw2, sw3):
    n = x_flat.shape[0]
    n_tiles = n // ROW_TILE
    return pl.pallas_call(
        _shared_body,
        grid=(n_tiles,),
        in_specs=[
            pl.BlockSpec((ROW_TILE, D_MODEL), lambda t: (t, 0)),
            pl.BlockSpec((D_MODEL, D_FF), lambda t: (0, 0)),
            pl.BlockSpec((D_MODEL, D_FF), lambda t: (0, 0)),
            pl.BlockSpec((D_FF, D_MODEL), lambda t: (0, 0)),
        ],
        out_specs=pl.BlockSpec((ROW_TILE, D_MODEL), lambda t: (t, 0)),
        out_shape=jax.ShapeDtypeStruct((n, D_MODEL), jnp.float32),
        compiler_params=pltpu.CompilerParams(
            dimension_semantics=("arbitrary",),
        ),
        interpret=_INTERPRET,
    )(x_flat, sw1, sw2, sw3)


def _wid():
    return lax.axis_index("s") * NC + lax.axis_index("c")


def _splat(v, k):
    return v[jnp.full((16,), k, jnp.int32)]


def _cumsum16(v, iota):
    for k in (1, 2, 4, 8):
        sv = v[jnp.maximum(iota - k, 0)]
        v = v + jnp.where(iota >= k, sv, 0)
    return v


def _make_hist_kernel(n_assign):
    chunk = n_assign // NW
    mesh = plsc.VectorSubcoreMesh(core_axis_name="c", subcore_axis_name="s")

    @functools.partial(
        pl.kernel,
        out_type=jax.ShapeDtypeStruct((NW, 16), jnp.int32),
        mesh=mesh,
        scratch_types=[
            pltpu.VMEM((chunk,), jnp.int32),
            pltpu.VMEM((16,), jnp.int32),
        ],
    )
    def hist_kernel(e_hbm, hist_hbm, ids_v, hist_v):
        w = _wid()
        pltpu.sync_copy(e_hbm.at[pl.ds(w * chunk, chunk)], ids_v)
        iota = lax.iota(jnp.int32, 16)
        accs = [jnp.zeros((16,), jnp.int32) for _ in range(NUM_E)]
        for j in range(chunk // 16):
            v = ids_v[pl.ds(j * 16, 16)]
            for e in range(NUM_E):
                accs[e] = accs[e] + jnp.where(v == e, 1, 0)
        hist = jnp.zeros((16,), jnp.int32)
        for e in range(NUM_E):
            tot_e = _splat(_cumsum16(accs[e], iota), 15)
            hist = jnp.where(iota == e, tot_e, hist)
        hist_v[...] = hist
        pltpu.sync_copy(hist_v, hist_hbm.at[w])

    return hist_kernel


def _make_sort_dispatch_kernel(n_tok, n_assign, te_len, pad_total):
    chunk = n_assign // NW
    rows = 32
    n_rows_chunks = chunk // rows
    mesh = plsc.VectorSubcoreMesh(core_axis_name="c", subcore_axis_name="s")

    @functools.partial(
        pl.kernel,
        out_type=[
            jax.ShapeDtypeStruct((n_assign // rows, rows), jnp.int32),
            jax.ShapeDtypeStruct((te_len,), jnp.int32),
            jax.ShapeDtypeStruct((te_len,), jnp.int32),
            jax.ShapeDtypeStruct((pad_total, D_MODEL), jnp.float32),
        ],
        mesh=mesh,
        scratch_types=[
            pltpu.VMEM((chunk,), jnp.int32),
            pltpu.VMEM((NW, 16), jnp.int32),
            pltpu.VMEM((n_rows_chunks, rows), jnp.int32),
            pltpu.VMEM((te_len,), jnp.int32),
            pltpu.VMEM((te_len,), jnp.int32),
            pltpu.VMEM((rows, D_MODEL), jnp.float32),
            pltpu.SemaphoreType.DMA,
        ],
    )
    def sort_dispatch_kernel(e_hbm, hist_hbm, x_hbm, dst_hbm, te_hbm, af_hbm,
                             disp_hbm, ids_v, allh_v, dst_v, te_v, af_v, rows_v, sem):
        w = _wid()
        pltpu.sync_copy(e_hbm.at[pl.ds(w * chunk, chunk)], ids_v)
        pltpu.sync_copy(hist_hbm, allh_v)
        iota = lax.iota(jnp.int32, 16)
        tot = jnp.zeros((16,), jnp.int32)
        pre = jnp.zeros((16,), jnp.int32)
        for ww in range(NW):
            row = allh_v[ww]
            tot = tot + row
            pre = pre + row * jnp.where(ww < w, 1, 0)
        padded = (tot + (ROW_TILE - 1)) & jnp.int32(-ROW_TILE)
        csum = _cumsum16(padded, iota)
        off = csum - padded
        start = off + pre

        cnt = jnp.zeros((16,), jnp.int32)
        for j in range(chunk // 16):
            v = ids_v[pl.ds(j * 16, 16)]
            rank = jnp.zeros((16,), jnp.int32)
            for k in range(1, 16):
                sv = v[jnp.maximum(iota - k, 0)]
                rank = rank + jnp.where((iota >= k) & (sv == v), 1, 0)
            dst_v[j // 2, pl.ds((j % 2) * 16, 16)] = start[v] + cnt[v] + rank
            cc = jnp.zeros((16,), jnp.int32)
            for k in range(16):
                cc = cc + jnp.where(iota == _splat(v, k), 1, 0)
            cnt = cnt + cc
        pltpu.sync_copy(dst_v, dst_hbm.at[pl.ds(w * n_rows_chunks, n_rows_chunks)])

        tok_per_slotblock = n_tok // NS
        for r in range(n_rows_chunks):
            tb = (w % NS) * tok_per_slotblock + r * rows
            pltpu.sync_copy(x_hbm.at[pl.ds(tb, rows)], rows_v)
            pltpu.async_copy(rows_v, disp_hbm.at[dst_v.at[r]], sem).wait()

        @pl.when(w == 0)
        def _():
            used_end = off + tot
            for g in range(te_len // 16):
                tstart = (iota + g * 16) * ROW_TILE
                acc = jnp.zeros((16,), jnp.int32)
                for e in range(NUM_E):
                    acc = acc + jnp.where(_splat(off, e) <= tstart, 1, 0)
                ex = acc - 1
                te_v[pl.ds(g * 16, 16)] = ex
                af_v[pl.ds(g * 16, 16)] = jnp.where(tstart < used_end[ex], 1, 0)
            pltpu.sync_copy(te_v, te_hbm)
            pltpu.sync_copy(af_v, af_hbm)

    return sort_dispatch_kernel


def _expert_ffn_body(te_ref, af_ref, x_ref, w1_ref, w2_ref, w3_ref, out_ref):
    del te_ref
    t = pl.program_id(0)

    @pl.when(af_ref[t] == 1)
    def _():
        x = x_ref[...]
        gate = _silu(jnp.dot(x, w1_ref[0], preferred_element_type=jnp.float32))
        up = jnp.dot(x, w2_ref[0], preferred_element_type=jnp.float32)
        out_ref[...] = jnp.dot(gate * up, w3_ref[0], preferred_element_type=jnp.float32)


def _expert_ffn(te, af, disp, ew1, ew2, ew3, n_tiles):
    pad_total = disp.shape[0]
    grid_spec = pltpu.PrefetchScalarGridSpec(
        num_scalar_prefetch=2,
        grid=(n_tiles,),
        in_specs=[
            pl.BlockSpec((ROW_TILE, D_MODEL), lambda t, te, af: (t, 0)),
            pl.BlockSpec((1, D_MODEL, D_FF), lambda t, te, af: (te[t], 0, 0)),
            pl.BlockSpec((1, D_MODEL, D_FF), lambda t, te, af: (te[t], 0, 0)),
            pl.BlockSpec((1, D_FF, D_MODEL), lambda t, te, af: (te[t], 0, 0)),
        ],
        out_specs=pl.BlockSpec((ROW_TILE, D_MODEL), lambda t, te, af: (t, 0)),
    )
    return pl.pallas_call(
        _expert_ffn_body,
        grid_spec=grid_spec,
        out_shape=jax.ShapeDtypeStruct((pad_total, D_MODEL), jnp.float32),
        compiler_params=pltpu.CompilerParams(
            dimension_semantics=("arbitrary",),
        ),
        interpret=_INTERPRET,
    )(te, af, disp, ew1, ew2, ew3)


def _make_gather_kernel(n_assign, pad_total):
    rows = 32
    n_rows_chunks = n_assign // rows // NW
    mesh = plsc.VectorSubcoreMesh(core_axis_name="c", subcore_axis_name="s")

    @functools.partial(
        pl.kernel,
        out_type=jax.ShapeDtypeStruct((n_assign, D_MODEL), jnp.float32),
        mesh=mesh,
        scratch_types=[
            pltpu.VMEM((rows, D_MODEL), jnp.float32),
            pltpu.VMEM((rows,), jnp.int32),
            pltpu.SemaphoreType.DMA,
        ],
    )
    def gather_kernel(eout_hbm, dst_hbm, g_hbm, rows_v, idx_v, sem):
        w = _wid()
        for c in range(n_rows_chunks):
            rr = w * n_rows_chunks + c
            pltpu.sync_copy(dst_hbm.at[rr], idx_v)
            pltpu.async_copy(eout_hbm.at[idx_v], rows_v, sem).wait()
            pltpu.sync_copy(rows_v, g_hbm.at[pl.ds(rr * rows, rows)])

    return gather_kernel


def _combine_body(sh_ref, g0_ref, g1_ref, w0_ref, w1_ref, out_ref):
    out_ref[...] = (sh_ref[...]
                    + w0_ref[...] * g0_ref[...]
                    + w1_ref[...] * g1_ref[...])


def _combine(shared_out, g, w0c, w1c, n_tok):
    n_tiles = n_tok // ROW_TILE
    return pl.pallas_call(
        _combine_body,
        grid=(n_tiles,),
        in_specs=[
            pl.BlockSpec((ROW_TILE, D_MODEL), lambda t: (t, 0)),
            pl.BlockSpec((ROW_TILE, D_MODEL), lambda t: (t, 0)),
            pl.BlockSpec((ROW_TILE, D_MODEL), lambda t: (t + n_tok // ROW_TILE, 0)),
            pl.BlockSpec((ROW_TILE, 1), lambda t: (t, 0)),
            pl.BlockSpec((ROW_TILE, 1), lambda t: (t, 0)),
        ],
        out_specs=pl.BlockSpec((ROW_TILE, D_MODEL), lambda t: (t, 0)),
        out_shape=jax.ShapeDtypeStruct((n_tok, D_MODEL), jnp.float32),
        compiler_params=pltpu.CompilerParams(
            dimension_semantics=("arbitrary",),
        ),
        interpret=_INTERPRET,
    )(shared_out, g, g, w0c, w1c)


def kernel(x, shared_w1, shared_w2, shared_w3, expert_w1, expert_w2, expert_w3, gate_w):
    Bn, Tn, C = x.shape
    n_tok = Bn * Tn
    n_assign = 2 * n_tok
    pad_total = n_assign + NUM_E * ROW_TILE
    n_tiles = n_assign // ROW_TILE + NUM_E
    te_len = 64

    x_flat = x.reshape(n_tok, C)

    a0, a1, w0c, w1c = _router(x_flat, gate_w)
    e_all = jnp.concatenate([a0.reshape(-1), a1.reshape(-1)])

    hist = _make_hist_kernel(n_assign)(e_all)
    dst2d, te, af, disp = _make_sort_dispatch_kernel(
        n_tok, n_assign, te_len, pad_total)(e_all, hist, x_flat)
    shared_out = _shared(x_flat, shared_w1, shared_w2, shared_w3)
    eout = _expert_ffn(te, af, disp, expert_w1, expert_w2, expert_w3, n_tiles)
    g = _make_gather_kernel(n_assign, pad_total)(eout, dst2d)
    out = _combine(shared_out, g, w0c, w1c, n_tok)

    final_out = out.reshape(Bn, Tn, C)
    aux_loss = jnp.array(0.0, dtype=jnp.float32)
    return (final_out, aux_loss)


# speedup vs baseline: 3.4771x; 1.0219x over previous
import functools

import jax
import jax.numpy as jnp
from jax import lax
from jax.experimental import pallas as pl
from jax.experimental.pallas import tpu as pltpu
from jax.experimental.pallas import tpu_sc as plsc

D_MODEL = 1024
D_FF = 2048
NUM_E = 8
ROW_TILE = 256

NC = 2
NS = 16
NW = NC * NS

_INTERPRET = False


def _silu(v):
    return v * (1.0 / (1.0 + jnp.exp(-v)))


def _router_body(x_ref, gate_ref, a0_ref, a1_ref, w0_ref, w1c_ref,
                 h0_ref, h1_ref):
    x = x_ref[...]
    rows = x.shape[0]
    logits = jnp.dot(x, gate_ref[...], preferred_element_type=jnp.float32)
    iota = lax.broadcasted_iota(jnp.int32, (rows, NUM_E), 1)
    m0 = jnp.max(logits, axis=1, keepdims=True)
    a0 = jnp.min(jnp.where(logits == m0, iota, NUM_E), axis=1, keepdims=True)
    masked = jnp.where(iota == a0, jnp.float32(-1e30), logits)
    m1 = jnp.max(masked, axis=1, keepdims=True)
    a1 = jnp.min(jnp.where(masked == m1, iota, NUM_E), axis=1, keepdims=True)
    t = jnp.exp(m1 - m0)
    a0_ref[...] = a0
    a1_ref[...] = a1
    w0_ref[...] = 1.0 / (1.0 + t)
    w1c_ref[...] = t / (1.0 + t)
    iota16 = lax.broadcasted_iota(jnp.int32, (rows, 16), 1)
    h0_ref[...] = jnp.sum(jnp.where(iota16 == a0, 1, 0),
                          axis=0, keepdims=True)[None]
    h1_ref[...] = jnp.sum(jnp.where(iota16 == a1, 1, 0),
                          axis=0, keepdims=True)[None]


def _router(x_flat, gate_w):
    n = x_flat.shape[0]
    n_tiles = n // ROW_TILE
    return pl.pallas_call(
        _router_body,
        grid=(n_tiles,),
        in_specs=[
            pl.BlockSpec((ROW_TILE, D_MODEL), lambda t: (t, 0)),
            pl.BlockSpec((D_MODEL, NUM_E), lambda t: (0, 0)),
        ],
        out_specs=[
            pl.BlockSpec((ROW_TILE, 1), lambda t: (t, 0)),
            pl.BlockSpec((ROW_TILE, 1), lambda t: (t, 0)),
            pl.BlockSpec((ROW_TILE, 1), lambda t: (t, 0)),
            pl.BlockSpec((ROW_TILE, 1), lambda t: (t, 0)),
            pl.BlockSpec((1, 1, 16), lambda t: (t, 0, 0)),
            pl.BlockSpec((1, 1, 16), lambda t: (t, 0, 0)),
        ],
        out_shape=[
            jax.ShapeDtypeStruct((n, 1), jnp.int32),
            jax.ShapeDtypeStruct((n, 1), jnp.int32),
            jax.ShapeDtypeStruct((n, 1), jnp.float32),
            jax.ShapeDtypeStruct((n, 1), jnp.float32),
            jax.ShapeDtypeStruct((n_tiles, 1, 16), jnp.int32),
            jax.ShapeDtypeStruct((n_tiles, 1, 16), jnp.int32),
        ],
        compiler_params=pltpu.CompilerParams(
            dimension_semantics=("arbitrary",),
        ),
        interpret=_INTERPRET,
    )(x_flat, gate_w)


def _shared_body(x_ref, w1_ref, w2_ref, w3_ref, sh_ref):
    x = x_ref[...]
    gate = _silu(jnp.dot(x, w1_ref[...], preferred_element_type=jnp.float32))
    up = jnp.dot(x, w2_ref[...], preferred_element_type=jnp.float32)
    sh_ref[...] = jnp.dot(gate * up, w3_ref[...], preferred_element_type=jnp.float32)


def _shared(x_flat, sw1, sw2, sw3):
    n = x_flat.shape[0]
    n_tiles = n // ROW_TILE
    return pl.pallas_call(
        _shared_body,
        grid=(n_tiles,),
        in_specs=[
            pl.BlockSpec((ROW_TILE, D_MODEL), lambda t: (t, 0)),
            pl.BlockSpec((D_MODEL, D_FF), lambda t: (0, 0)),
            pl.BlockSpec((D_MODEL, D_FF), lambda t: (0, 0)),
            pl.BlockSpec((D_FF, D_MODEL), lambda t: (0, 0)),
        ],
        out_specs=pl.BlockSpec((ROW_TILE, D_MODEL), lambda t: (t, 0)),
        out_shape=jax.ShapeDtypeStruct((n, D_MODEL), jnp.float32),
        compiler_params=pltpu.CompilerParams(
            dimension_semantics=("arbitrary",),
        ),
        interpret=_INTERPRET,
    )(x_flat, sw1, sw2, sw3)


def _wid():
    return lax.axis_index("s") * NC + lax.axis_index("c")


def _splat(v, k):
    return v[jnp.full((16,), k, jnp.int32)]


def _cumsum16(v, iota):
    for k in (1, 2, 4, 8):
        sv = v[jnp.maximum(iota - k, 0)]
        v = v + jnp.where(iota >= k, sv, 0)
    return v


def _make_sort_dispatch_kernel(n_tok, n_assign, te_len, pad_total):
    chunk = n_assign // NW
    rows = 32
    n_rows_chunks = chunk // rows
    mesh = plsc.VectorSubcoreMesh(core_axis_name="c", subcore_axis_name="s")

    @functools.partial(
        pl.kernel,
        out_type=[
            jax.ShapeDtypeStruct((n_assign // rows, rows), jnp.int32),
            jax.ShapeDtypeStruct((te_len,), jnp.int32),
            jax.ShapeDtypeStruct((te_len,), jnp.int32),
            jax.ShapeDtypeStruct((pad_total, D_MODEL), jnp.float32),
        ],
        mesh=mesh,
        scratch_types=[
            pltpu.VMEM((chunk,), jnp.int32),
            pltpu.VMEM((NW, 16), jnp.int32),
            pltpu.VMEM((n_rows_chunks, rows), jnp.int32),
            pltpu.VMEM((te_len,), jnp.int32),
            pltpu.VMEM((te_len,), jnp.int32),
            pltpu.VMEM((rows, D_MODEL), jnp.float32),
            pltpu.VMEM((rows, D_MODEL), jnp.float32),
            pltpu.SemaphoreType.DMA,
            pltpu.SemaphoreType.DMA,
        ],
    )
    def sort_dispatch_kernel(e_hbm, hist_hbm, x_hbm, dst_hbm, te_hbm, af_hbm,
                             disp_hbm, ids_v, allh_v, dst_v, te_v, af_v,
                             rows_v0, rows_v1, sem0, sem1):
        w = _wid()
        pltpu.sync_copy(e_hbm.at[pl.ds(w * chunk, chunk)], ids_v)
        pltpu.sync_copy(hist_hbm, allh_v)
        iota = lax.iota(jnp.int32, 16)
        tot = jnp.zeros((16,), jnp.int32)
        pre = jnp.zeros((16,), jnp.int32)
        for ww in range(NW):
            row = allh_v[ww]
            tot = tot + row
            pre = pre + row * jnp.where(ww < w, 1, 0)
        padded = (tot + (ROW_TILE - 1)) & jnp.int32(-ROW_TILE)
        csum = _cumsum16(padded, iota)
        off = csum - padded
        start = off + pre

        cnt = jnp.zeros((16,), jnp.int32)
        for j in range(chunk // 16):
            v = ids_v[pl.ds(j * 16, 16)]
            rank = jnp.zeros((16,), jnp.int32)
            for k in range(1, 16):
                sv = v[jnp.maximum(iota - k, 0)]
                rank = rank + jnp.where((iota >= k) & (sv == v), 1, 0)
            dst_v[j // 2, pl.ds((j % 2) * 16, 16)] = start[v] + cnt[v] + rank
            cc = jnp.zeros((16,), jnp.int32)
            for k in range(16):
                cc = cc + jnp.where(iota == _splat(v, k), 1, 0)
            cnt = cnt + cc
        pltpu.sync_copy(dst_v, dst_hbm.at[pl.ds(w * n_rows_chunks, n_rows_chunks)])

        tok_per_slotblock = n_tok // NS
        bufs = (rows_v0, rows_v1)
        sems = (sem0, sem1)
        cps = [None, None]
        for r in range(n_rows_chunks):
            b = r % 2
            if cps[b] is not None:
                cps[b].wait()
            tb = (w % NS) * tok_per_slotblock + r * rows
            pltpu.sync_copy(x_hbm.at[pl.ds(tb, rows)], bufs[b])
            cps[b] = pltpu.async_copy(bufs[b], disp_hbm.at[dst_v.at[r]], sems[b])
        for cp in cps:
            if cp is not None:
                cp.wait()

        @pl.when(w == 0)
        def _():
            used_end = off + tot
            for g in range(te_len // 16):
                tstart = (iota + g * 16) * ROW_TILE
                acc = jnp.zeros((16,), jnp.int32)
                for e in range(NUM_E):
                    acc = acc + jnp.where(_splat(off, e) <= tstart, 1, 0)
                ex = acc - 1
                te_v[pl.ds(g * 16, 16)] = ex
                af_v[pl.ds(g * 16, 16)] = jnp.where(tstart < used_end[ex], 1, 0)
            pltpu.sync_copy(te_v, te_hbm)
            pltpu.sync_copy(af_v, af_hbm)

    return sort_dispatch_kernel


def _expert_ffn_body(te_ref, af_ref, x_ref, w1_ref, w2_ref, w3_ref, out_ref):
    del te_ref
    t = pl.program_id(0)

    @pl.when(af_ref[t] == 1)
    def _():
        x = x_ref[...]
        gate = _silu(jnp.dot(x, w1_ref[0], preferred_element_type=jnp.float32))
        up = jnp.dot(x, w2_ref[0], preferred_element_type=jnp.float32)
        out_ref[...] = jnp.dot(gate * up, w3_ref[0], preferred_element_type=jnp.float32)


def _expert_ffn(te, af, disp, ew1, ew2, ew3, n_tiles):
    pad_total = disp.shape[0]
    grid_spec = pltpu.PrefetchScalarGridSpec(
        num_scalar_prefetch=2,
        grid=(n_tiles,),
        in_specs=[
            pl.BlockSpec((ROW_TILE, D_MODEL), lambda t, te, af: (t, 0)),
            pl.BlockSpec((1, D_MODEL, D_FF), lambda t, te, af: (te[t], 0, 0)),
            pl.BlockSpec((1, D_MODEL, D_FF), lambda t, te, af: (te[t], 0, 0)),
            pl.BlockSpec((1, D_FF, D_MODEL), lambda t, te, af: (te[t], 0, 0)),
        ],
        out_specs=pl.BlockSpec((ROW_TILE, D_MODEL), lambda t, te, af: (t, 0)),
    )
    return pl.pallas_call(
        _expert_ffn_body,
        grid_spec=grid_spec,
        out_shape=jax.ShapeDtypeStruct((pad_total, D_MODEL), jnp.float32),
        compiler_params=pltpu.CompilerParams(
            dimension_semantics=("arbitrary",),
        ),
        interpret=_INTERPRET,
    )(te, af, disp, ew1, ew2, ew3)


def _make_gather_kernel(n_assign, pad_total):
    rows = 32
    n_rows_chunks = n_assign // rows // NW
    mesh = plsc.VectorSubcoreMesh(core_axis_name="c", subcore_axis_name="s")

    @functools.partial(
        pl.kernel,
        out_type=jax.ShapeDtypeStruct((n_assign, D_MODEL), jnp.float32),
        mesh=mesh,
        scratch_types=[
            pltpu.VMEM((rows, D_MODEL), jnp.float32),
            pltpu.VMEM((rows, D_MODEL), jnp.float32),
            pltpu.VMEM((n_rows_chunks, rows), jnp.int32),
            pltpu.SemaphoreType.DMA,
            pltpu.SemaphoreType.DMA,
        ],
    )
    def gather_kernel(eout_hbm, dst_hbm, g_hbm, rows_v0, rows_v1, idx_v,
                      sem0, sem1):
        w = _wid()
        pltpu.sync_copy(
            dst_hbm.at[pl.ds(w * n_rows_chunks, n_rows_chunks)], idx_v)
        bufs = (rows_v0, rows_v1)
        sems = (sem0, sem1)
        cps = [None, None]
        for c in range(n_rows_chunks):
            b = c % 2
            cps[b] = pltpu.async_copy(eout_hbm.at[idx_v.at[c]], bufs[b], sems[b])
            if c > 0:
                cps[1 - b].wait()
                rr = w * n_rows_chunks + c - 1
                pltpu.sync_copy(bufs[1 - b], g_hbm.at[pl.ds(rr * rows, rows)])
        last = n_rows_chunks - 1
        cps[last % 2].wait()
        pltpu.sync_copy(bufs[last % 2],
                        g_hbm.at[pl.ds((w * n_rows_chunks + last) * rows, rows)])

    return gather_kernel


def _combine_body(sh_ref, g0_ref, g1_ref, w0_ref, w1_ref, out_ref):
    out_ref[...] = (sh_ref[...]
                    + w0_ref[...] * g0_ref[...]
                    + w1_ref[...] * g1_ref[...])


def _combine(shared_out, g, w0c, w1c, n_tok):
    n_tiles = n_tok // ROW_TILE
    return pl.pallas_call(
        _combine_body,
        grid=(n_tiles,),
        in_specs=[
            pl.BlockSpec((ROW_TILE, D_MODEL), lambda t: (t, 0)),
            pl.BlockSpec((ROW_TILE, D_MODEL), lambda t: (t, 0)),
            pl.BlockSpec((ROW_TILE, D_MODEL), lambda t: (t + n_tok // ROW_TILE, 0)),
            pl.BlockSpec((ROW_TILE, 1), lambda t: (t, 0)),
            pl.BlockSpec((ROW_TILE, 1), lambda t: (t, 0)),
        ],
        out_specs=pl.BlockSpec((ROW_TILE, D_MODEL), lambda t: (t, 0)),
        out_shape=jax.ShapeDtypeStruct((n_tok, D_MODEL), jnp.float32),
        compiler_params=pltpu.CompilerParams(
            dimension_semantics=("arbitrary",),
        ),
        interpret=_INTERPRET,
    )(shared_out, g, g, w0c, w1c)


def kernel(x, shared_w1, shared_w2, shared_w3, expert_w1, expert_w2, expert_w3, gate_w):
    Bn, Tn, C = x.shape
    n_tok = Bn * Tn
    n_assign = 2 * n_tok
    pad_total = n_assign + NUM_E * ROW_TILE
    n_tiles = n_assign // ROW_TILE + NUM_E
    te_len = 64

    x_flat = x.reshape(n_tok, C)

    a0, a1, w0c, w1c, h0, h1 = _router(x_flat, gate_w)
    e_all = jnp.concatenate([a0.reshape(-1), a1.reshape(-1)])
    hist = jnp.concatenate([h0.reshape(NS, 16), h1.reshape(NS, 16)], axis=0)

    dst2d, te, af, disp = _make_sort_dispatch_kernel(
        n_tok, n_assign, te_len, pad_total)(e_all, hist, x_flat)
    shared_out = _shared(x_flat, shared_w1, shared_w2, shared_w3)
    eout = _expert_ffn(te, af, disp, expert_w1, expert_w2, expert_w3, n_tiles)
    g = _make_gather_kernel(n_assign, pad_total)(eout, dst2d)
    out = _combine(shared_out, g, w0c, w1c, n_tok)

    final_out = out.reshape(Bn, Tn, C)
    aux_loss = jnp.array(0.0, dtype=jnp.float32)
    return (final_out, aux_loss)
